# Initial kernel scaffold; baseline (speedup 1.0000x reference)
import jax
import jax.numpy as jnp
from jax.experimental import pallas as pl


def kernel(token_ids, table):
    raise NotImplementedError("write your pallas kernel here")


# speedup vs baseline: 6.8532x; 6.8532x over previous
import functools

import jax
import jax.numpy as jnp
from jax import lax
from jax.experimental import pallas as pl
from jax.experimental.pallas import tpu as pltpu
from jax.experimental.pallas import tpu_sc as plsc

_LANE = 128


def _emb_lookup(idx2, table, *, n_workers, k_per_chunk, rows_per_w):
    NR = idx2.shape[0]
    D = table.shape[1]
    B = NR * _LANE
    n_chunks = rows_per_w // k_per_chunk
    K = k_per_chunk

    mesh = plsc.VectorSubcoreMesh(core_axis_name="c", subcore_axis_name="s")

    @functools.partial(
        pl.kernel,
        out_type=jax.ShapeDtypeStruct((B, D), jnp.float32),
        mesh=mesh,
        scratch_types=[
            pltpu.VMEM((K, _LANE), jnp.int32),
            pltpu.VMEM((K * _LANE, D), jnp.float32),
            pltpu.SemaphoreType.DMA,
        ],
    )
    def emb(idx_hbm, tab_hbm, out_hbm, idx_v, rows_v, sem):
        wid = lax.axis_index("s") * 2 + lax.axis_index("c")
        r0 = wid * rows_per_w

        @pl.loop(0, n_chunks)
        def chunk(c):
            row = r0 + c * K
            pltpu.sync_copy(idx_hbm.at[pl.ds(row, K)], idx_v)
            cps = [
                pltpu.async_copy(
                    tab_hbm.at[idx_v.at[j]],
                    rows_v.at[pl.ds(j * _LANE, _LANE)],
                    sem,
                )
                for j in range(K)
            ]
            for cp in cps:
                cp.wait()
            pltpu.sync_copy(rows_v, out_hbm.at[pl.ds(row * _LANE, K * _LANE)])

    return emb(idx2, table)


def kernel(token_ids, table):
    B0, S = token_ids.shape
    V, D = table.shape
    B = B0 * S
    idx = token_ids.reshape(B).astype(jnp.int32)
    NR = B // _LANE
    NW = 32
    rows_per_w = NR // NW
    idx2 = idx.reshape(NR, _LANE)
    out = _emb_lookup(idx2, table, n_workers=NW, k_per_chunk=2,
                      rows_per_w=rows_per_w)
    return out.reshape(B0, S, D)


# speedup vs baseline: 9.0900x; 1.3264x over previous
import functools

import jax
import jax.numpy as jnp
from jax import lax
from jax.experimental import pallas as pl
from jax.experimental.pallas import tpu as pltpu
from jax.experimental.pallas import tpu_sc as plsc

_LANE = 128
_NBUF = 4


def _emb_lookup(idx2, table, *, rows_per_w):
    NR = idx2.shape[0]
    D = table.shape[1]
    B = NR * _LANE
    n_groups = rows_per_w // _NBUF

    mesh = plsc.VectorSubcoreMesh(core_axis_name="c", subcore_axis_name="s")

    @functools.partial(
        pl.kernel,
        out_type=jax.ShapeDtypeStruct((B, D), jnp.float32),
        mesh=mesh,
        scratch_types=[
            pltpu.VMEM((rows_per_w, _LANE), jnp.int32),
            pltpu.VMEM((_NBUF, _LANE, D), jnp.float32),
            pltpu.SemaphoreType.DMA((_NBUF,)),
            pltpu.SemaphoreType.DMA((_NBUF,)),
        ],
    )
    def emb(idx_hbm, tab_hbm, out_hbm, idx_v, rows_v, gsem, osem):
        wid = lax.axis_index("s") * 2 + lax.axis_index("c")
        r0 = wid * rows_per_w

        pltpu.sync_copy(idx_hbm.at[pl.ds(r0, rows_per_w)], idx_v)

        def g_fire(c, b):
            pltpu.async_copy(tab_hbm.at[idx_v.at[c]], rows_v.at[b],
                             gsem.at[b])

        def g_wait(c, b):
            pltpu.make_async_copy(tab_hbm.at[idx_v.at[c]], rows_v.at[b],
                                  gsem.at[b]).wait()

        def w_fire(c, b):
            pltpu.async_copy(rows_v.at[b],
                             out_hbm.at[pl.ds((r0 + c) * _LANE, _LANE)],
                             osem.at[b])

        def w_wait(c, b):
            pltpu.make_async_copy(rows_v.at[b],
                                  out_hbm.at[pl.ds((r0 + c) * _LANE, _LANE)],
                                  osem.at[b]).wait()

        for b in range(_NBUF):
            g_fire(b, b)

        @pl.loop(0, n_groups)
        def grp(g):
            c0 = g * _NBUF
            for b in range(_NBUF):
                g_wait(c0 + b, b)
                w_fire(c0 + b, b)
            for b in range(_NBUF):
                cn = c0 + _NBUF + b

                @pl.when(cn < rows_per_w)
                def _():
                    w_wait(cn - _NBUF, b)
                    g_fire(cn, b)

        for b in range(_NBUF):
            w_wait(rows_per_w - _NBUF + b, b)

    return emb(idx2, table)


def kernel(token_ids, table):
    B0, S = token_ids.shape
    V, D = table.shape
    B = B0 * S
    idx = token_ids.reshape(B).astype(jnp.int32)
    NR = B // _LANE
    NW = 32
    rows_per_w = NR // NW
    idx2 = idx.reshape(NR, _LANE)
    out = _emb_lookup(idx2, table, rows_per_w=rows_per_w)
    return out.reshape(B0, S, D)


# speedup vs baseline: 9.1267x; 1.0040x over previous
import functools

import jax
import jax.numpy as jnp
from jax import lax
from jax.experimental import pallas as pl
from jax.experimental.pallas import tpu as pltpu
from jax.experimental.pallas import tpu_sc as plsc

_LANE = 128
_NBUF = 5


def _emb_lookup(idx2, table, *, rows_per_w):
    NR = idx2.shape[0]
    D = table.shape[1]
    B = NR * _LANE
    n_groups = rows_per_w // _NBUF

    mesh = plsc.VectorSubcoreMesh(core_axis_name="c", subcore_axis_name="s")

    @functools.partial(
        pl.kernel,
        out_type=jax.ShapeDtypeStruct((B, D), jnp.float32),
        mesh=mesh,
        scratch_types=[
            pltpu.VMEM((rows_per_w, _LANE), jnp.int32),
            pltpu.VMEM((_NBUF, _LANE, D), jnp.float32),
            pltpu.SemaphoreType.DMA((_NBUF,)),
            pltpu.SemaphoreType.DMA((_NBUF,)),
        ],
    )
    def emb(idx_hbm, tab_hbm, out_hbm, idx_v, rows_v, gsem, osem):
        wid = lax.axis_index("s") * 2 + lax.axis_index("c")
        r0 = wid * rows_per_w

        pltpu.sync_copy(idx_hbm.at[pl.ds(r0, rows_per_w)], idx_v)

        def g_fire(c, b):
            pltpu.async_copy(tab_hbm.at[idx_v.at[c]], rows_v.at[b],
                             gsem.at[b])

        def g_wait(c, b):
            pltpu.make_async_copy(tab_hbm.at[idx_v.at[c]], rows_v.at[b],
                                  gsem.at[b]).wait()

        def w_fire(c, b):
            pltpu.async_copy(rows_v.at[b],
                             out_hbm.at[pl.ds((r0 + c) * _LANE, _LANE)],
                             osem.at[b])

        def w_wait(c, b):
            pltpu.make_async_copy(rows_v.at[b],
                                  out_hbm.at[pl.ds((r0 + c) * _LANE, _LANE)],
                                  osem.at[b]).wait()

        for b in range(_NBUF):
            g_fire(b, b)

        @pl.loop(0, n_groups)
        def grp(g):
            c0 = g * _NBUF
            for b in range(_NBUF):
                g_wait(c0 + b, b)
                w_fire(c0 + b, b)
            for b in range(_NBUF):
                cn = c0 + _NBUF + b

                @pl.when(cn < rows_per_w)
                def _():
                    w_wait(cn - _NBUF, b)
                    g_fire(cn, b)

        for b in range(_NBUF):
            w_wait(rows_per_w - _NBUF + b, b)

    return emb(idx2, table)


def kernel(token_ids, table):
    B0, S = token_ids.shape
    V, D = table.shape
    B = B0 * S
    idx = token_ids.reshape(B).astype(jnp.int32)
    NR = B // _LANE
    NW = 32
    rows_per_w = NR // NW
    idx2 = idx.reshape(NR, _LANE)
    out = _emb_lookup(idx2, table, rows_per_w=rows_per_w)
    return out.reshape(B0, S, D)


# speedup vs baseline: 9.1551x; 1.0031x over previous
import functools

import jax
import jax.numpy as jnp
from jax import lax
from jax.experimental import pallas as pl
from jax.experimental.pallas import tpu as pltpu
from jax.experimental.pallas import tpu_sc as plsc

_LANE = 128
_K = 2
_NBUF = 3


def _emb_lookup(idx2, table, *, rows_per_w):
    NR = idx2.shape[0]
    D = table.shape[1]
    B = NR * _LANE
    n_sc = rows_per_w // _K
    n_groups = n_sc // _NBUF
    n_tail = n_sc - n_groups * _NBUF

    mesh = plsc.VectorSubcoreMesh(core_axis_name="c", subcore_axis_name="s")

    @functools.partial(
        pl.kernel,
        out_type=jax.ShapeDtypeStruct((B, D), jnp.float32),
        mesh=mesh,
        scratch_types=[
            pltpu.VMEM((rows_per_w, _LANE), jnp.int32),
            pltpu.VMEM((_NBUF, _K * _LANE, D), jnp.float32),
            pltpu.SemaphoreType.DMA((_NBUF,)),
            pltpu.SemaphoreType.DMA((_NBUF,)),
        ],
    )
    def emb(idx_hbm, tab_hbm, out_hbm, idx_v, rows_v, gsem, osem):
        wid = lax.axis_index("s") * 2 + lax.axis_index("c")
        r0 = wid * rows_per_w

        pltpu.sync_copy(idx_hbm.at[pl.ds(r0, rows_per_w)], idx_v)

        def g_fire(c, b):
            for j in range(_K):
                pltpu.async_copy(
                    tab_hbm.at[idx_v.at[c * _K + j]],
                    rows_v.at[b, pl.ds(j * _LANE, _LANE)],
                    gsem.at[b])

        def g_wait(c, b):
            for j in range(_K):
                pltpu.make_async_copy(
                    tab_hbm.at[idx_v.at[c * _K + j]],
                    rows_v.at[b, pl.ds(j * _LANE, _LANE)],
                    gsem.at[b]).wait()

        def w_fire(c, b):
            pltpu.async_copy(
                rows_v.at[b],
                out_hbm.at[pl.ds((r0 + c * _K) * _LANE, _K * _LANE)],
                osem.at[b])

        def w_wait(c, b):
            pltpu.make_async_copy(
                rows_v.at[b],
                out_hbm.at[pl.ds((r0 + c * _K) * _LANE, _K * _LANE)],
                osem.at[b]).wait()

        for b in range(_NBUF):
            g_fire(b, b)

        @pl.loop(0, n_groups)
        def grp(g):
            c0 = g * _NBUF
            for b in range(_NBUF):
                g_wait(c0 + b, b)
                w_fire(c0 + b, b)
            for b in range(_NBUF):
                cn = c0 + _NBUF + b

                @pl.when(cn < n_sc)
                def _():
                    w_wait(cn - _NBUF, b)
                    g_fire(cn, b)

        for b in range(n_tail):
            c = n_groups * _NBUF + b
            g_wait(c, b)
            w_fire(c, b)
        for b in range(n_tail):
            w_wait(n_groups * _NBUF + b, b)
        for b in range(_NBUF - n_tail):
            w_wait(n_groups * _NBUF - _NBUF + n_tail + b, n_tail + b)

    return emb(idx2, table)


def kernel(token_ids, table):
    B0, S = token_ids.shape
    V, D = table.shape
    B = B0 * S
    idx = token_ids.reshape(B).astype(jnp.int32)
    NR = B // _LANE
    NW = 32
    rows_per_w = NR // NW
    idx2 = idx.reshape(NR, _LANE)
    out = _emb_lookup(idx2, table, rows_per_w=rows_per_w)
    return out.reshape(B0, S, D)


# speedup vs baseline: 9.1590x; 1.0004x over previous
import functools

import jax
import jax.numpy as jnp
from jax import lax
from jax.experimental import pallas as pl
from jax.experimental.pallas import tpu as pltpu
from jax.experimental.pallas import tpu_sc as plsc

_LANE = 128
_K = 2
_NBUF = 3


def _emb_lookup(idx2, table, *, rows_per_w, n_cores):
    NR = idx2.shape[0]
    D = table.shape[1]
    B = NR * _LANE
    n_sc = rows_per_w // _K
    n_groups = n_sc // _NBUF
    n_tail = n_sc - n_groups * _NBUF

    mesh = plsc.VectorSubcoreMesh(core_axis_name="c", subcore_axis_name="s")

    @functools.partial(
        pl.kernel,
        out_type=jax.ShapeDtypeStruct((B, D), jnp.float32),
        mesh=mesh,
        scratch_types=[
            pltpu.VMEM((rows_per_w, _LANE), jnp.int32),
            pltpu.VMEM((_NBUF, _K * _LANE, D), jnp.float32),
            pltpu.SemaphoreType.DMA((_NBUF,)),
            pltpu.SemaphoreType.DMA((_NBUF,)),
        ],
    )
    def emb(idx_hbm, tab_hbm, out_hbm, idx_v, rows_v, gsem, osem):
        wid = lax.axis_index("s") * n_cores + lax.axis_index("c")
        r0 = wid * rows_per_w

        pltpu.sync_copy(idx_hbm.at[pl.ds(r0, rows_per_w)], idx_v)

        def g_fire(c, b):
            for j in range(_K):
                pltpu.async_copy(
                    tab_hbm.at[idx_v.at[c * _K + j]],
                    rows_v.at[b, pl.ds(j * _LANE, _LANE)],
                    gsem.at[b])

        def g_wait(c, b):
            for j in range(_K):
                pltpu.make_async_copy(
                    tab_hbm.at[idx_v.at[c * _K + j]],
                    rows_v.at[b, pl.ds(j * _LANE, _LANE)],
                    gsem.at[b]).wait()

        def w_fire(c, b):
            pltpu.async_copy(
                rows_v.at[b],
                out_hbm.at[pl.ds((r0 + c * _K) * _LANE, _K * _LANE)],
                osem.at[b])

        def w_wait(c, b):
            pltpu.make_async_copy(
                rows_v.at[b],
                out_hbm.at[pl.ds((r0 + c * _K) * _LANE, _K * _LANE)],
                osem.at[b]).wait()

        for b in range(_NBUF):
            g_fire(b, b)

        @pl.loop(0, n_groups)
        def grp(g):
            c0 = g * _NBUF
            for b in range(_NBUF):
                g_wait(c0 + b, b)
                w_fire(c0 + b, b)
            for b in range(_NBUF):
                cn = c0 + _NBUF + b

                @pl.when(cn < n_sc)
                def _():
                    w_wait(cn - _NBUF, b)
                    g_fire(cn, b)

        for b in range(n_tail):
            c = n_groups * _NBUF + b
            g_wait(c, b)
            w_fire(c, b)
        for b in range(n_tail):
            w_wait(n_groups * _NBUF + b, b)
        for b in range(_NBUF - n_tail):
            w_wait(n_groups * _NBUF - _NBUF + n_tail + b, n_tail + b)

    return emb(idx2, table)


def kernel(token_ids, table):
    B0, S = token_ids.shape
    V, D = table.shape
    B = B0 * S
    idx = token_ids.reshape(B).astype(jnp.int32)
    NR = B // _LANE
    info = plsc.get_sparse_core_info()
    NW = info.num_cores * info.num_subcores
    rows_per_w = NR // NW
    idx2 = idx.reshape(NR, _LANE)
    out = _emb_lookup(idx2, table, rows_per_w=rows_per_w,
                      n_cores=info.num_cores)
    return out.reshape(B0, S, D)
